# Initial kernel scaffold; baseline (speedup 1.0000x reference)
import jax
import jax.numpy as jnp
from jax.experimental import pallas as pl


def kernel(h1, h2, h3, h4, e_up_2_0, e_up_2_1, e_up_3_0, e_up_3_1, e_up_4_0, e_up_4_1, e_down_4_0, e_down_4_1, e_down_3_0, e_down_3_1, e_down_2_0, e_down_2_1, up2_W1, up2_b1, up2_W2, up2_b2, up3_W1, up3_b1, up3_W2, up3_b2, up4_W1, up4_b1, up4_W2, up4_b2, dn1_W1, dn1_b1, dn1_W2, dn1_b2, dn2_W1, dn2_b1, dn2_W2, dn2_b2, dn3_W1, dn3_b1, dn3_W2, dn3_b2):
    raise NotImplementedError("write your pallas kernel here")


# speedup vs baseline: 7.7611x; 7.7611x over previous
import functools

import jax
import jax.numpy as jnp
from jax import lax
from jax.experimental import pallas as pl
from jax.experimental.pallas import tpu as pltpu
from jax.experimental.pallas import tpu_sc as plsc

U = 32
NW = 32
ZC = 32
CHUNK = 256


MB = 2048


def _pad_rows(n):
    return ((n + MB - 1) // MB) * MB


@functools.cache
def _make_agg(nsrc, rp, etot):
    epw = etot // NW
    nch = epw // CHUNK
    rpt = rp // 16
    nz = rpt // ZC
    mesh = plsc.VectorSubcoreMesh(core_axis_name="c", subcore_axis_name="s")

    @functools.partial(
        pl.kernel,
        out_type=jax.ShapeDtypeStruct((2, rp, U), jnp.float32),
        mesh=mesh,
        compiler_params=pltpu.CompilerParams(use_tc_tiling_on_sc=False),
        scratch_types=[
            pltpu.VMEM_SHARED((rp, U), jnp.float32),
            pltpu.VMEM((ZC, U), jnp.float32),
            pltpu.VMEM((2, CHUNK), jnp.int32),
            pltpu.VMEM((2, CHUNK), jnp.int32),
            pltpu.VMEM((CHUNK, U), jnp.float32),
            pltpu.VMEM((CHUNK, U), jnp.float32),
            pltpu.SemaphoreType.DMA,
            pltpu.SemaphoreType.DMA,
            pltpu.SemaphoreType.DMA,
            pltpu.SemaphoreType.DMA,
        ],
    )
    def agg(table, src, dst, out, acc, zbuf, src_v, dst_v, rows0, rows1,
            g0, g1, is0, is1):
        c = lax.axis_index("c")
        s = lax.axis_index("s")
        wid = c * 16 + s
        def zb(i, carry):
            zbuf[i, pl.ds(0, 16)] = jnp.zeros((16,), jnp.float32)
            zbuf[i, pl.ds(16, 16)] = jnp.zeros((16,), jnp.float32)
            return carry
        lax.fori_loop(0, ZC, zb, 0)
        base_r = s * rpt
        def zacc(k, carry):
            pltpu.sync_copy(zbuf, acc.at[pl.ds(base_r + k * ZC, ZC)])
            return carry
        lax.fori_loop(0, nz, zacc, 0)
        plsc.subcore_barrier()
        eb = wid * epw
        rows = (rows0, rows1)
        gsem = (g0, g1)
        ism = (is0, is1)

        def load_idx(j):
            b = j % 2
            hs_ = pltpu.async_copy(src.at[pl.ds(eb + j * CHUNK, CHUNK)],
                                   src_v.at[b], ism[b])
            hd_ = pltpu.async_copy(dst.at[pl.ds(eb + j * CHUNK, CHUNK)],
                                   dst_v.at[b], ism[b])
            return hs_, hd_

        gh = [None] * nch
        ih = [None] * nch
        ih[0] = load_idx(0)
        ih[0][0].wait()
        gh[0] = pltpu.async_copy(table.at[src_v.at[0]], rows0, g0)
        if nch > 1:
            ih[1] = load_idx(1)
        for j in range(nch):
            b = j % 2
            nb = (j + 1) % 2
            gh[j].wait()
            if j + 1 < nch:
                ih[j + 1][0].wait()
                gh[j + 1] = pltpu.async_copy(table.at[src_v.at[nb]],
                                             rows[nb], gsem[nb])
            ih[j][1].wait()
            pltpu.sync_copy(rows[b], acc.at[dst_v.at[b]], add=True)
            if j + 2 < nch:
                ih[j + 2] = load_idx(j + 2)
        plsc.subcore_barrier()
        pltpu.sync_copy(acc.at[pl.ds(base_r, rpt)],
                        out.at[c, pl.ds(base_r, rpt)])

    return agg


def _mlp_up(h, parts, w1, b1, w2, b2, n, ngp):
    def body(h_ref, p0_ref, p1_ref, w1_ref, b1_ref, w2_ref, b2_ref, o_ref):
        hv = h_ref[...]
        a0 = p0_ref[0] + p0_ref[1]
        a1 = p1_ref[0] + p1_ref[1]
        x1 = (jnp.dot(hv, w1_ref[0:U, :], preferred_element_type=jnp.float32)
              + jnp.dot(a0, w1_ref[U:2 * U, :], preferred_element_type=jnp.float32)
              + jnp.dot(a1, w1_ref[2 * U:3 * U, :], preferred_element_type=jnp.float32)
              + b1_ref[...][None, :])
        t = jnp.tanh(x1)
        o_ref[...] = (jnp.dot(t, w2_ref[...], preferred_element_type=jnp.float32)
                      + b2_ref[...][None, :])
    off = ngp // MB
    return pl.pallas_call(
        body,
        grid=(pl.cdiv(n, MB),),
        in_specs=[
            pl.BlockSpec((MB, U), lambda i: (i, 0)),
            pl.BlockSpec((2, MB, U), lambda i: (0, i, 0)),
            pl.BlockSpec((2, MB, U), lambda i: (0, off + i, 0)),
            pl.BlockSpec((3 * U, U), lambda i: (0, 0)),
            pl.BlockSpec((U,), lambda i: (0,)),
            pl.BlockSpec((U, U), lambda i: (0, 0)),
            pl.BlockSpec((U,), lambda i: (0,)),
        ],
        out_specs=pl.BlockSpec((MB, U), lambda i: (i, 0)),
        out_shape=jax.ShapeDtypeStruct((n, U), jnp.float32),
    )(h, parts, parts, w1, b1, w2, b2)


def _mlp_dn(h, parts, w1, b1, w2, b2, n):
    def body(h_ref, p_ref, w1_ref, b1_ref, w2_ref, b2_ref, o_ref):
        hv = h_ref[...]
        hd = p_ref[0] + p_ref[1]
        x1 = (jnp.dot(hv, w1_ref[0:U, :], preferred_element_type=jnp.float32)
              + jnp.dot(hd, w1_ref[U:2 * U, :], preferred_element_type=jnp.float32)
              + b1_ref[...][None, :])
        t = jnp.tanh(x1)
        o_ref[...] = (jnp.dot(t, w2_ref[...], preferred_element_type=jnp.float32)
                      + b2_ref[...][None, :])
    return pl.pallas_call(
        body,
        grid=(pl.cdiv(n, MB),),
        in_specs=[
            pl.BlockSpec((MB, U), lambda i: (i, 0)),
            pl.BlockSpec((2, MB, U), lambda i: (0, i, 0)),
            pl.BlockSpec((2 * U, U), lambda i: (0, 0)),
            pl.BlockSpec((U,), lambda i: (0,)),
            pl.BlockSpec((U, U), lambda i: (0, 0)),
            pl.BlockSpec((U,), lambda i: (0,)),
        ],
        out_specs=pl.BlockSpec((MB, U), lambda i: (i, 0)),
        out_shape=jax.ShapeDtypeStruct((n, U), jnp.float32),
    )(h, parts, w1, b1, w2, b2)


def _pad_edges(ids, dump, epad):
    pad = epad - ids.shape[0]
    if pad == 0:
        return ids
    return jnp.concatenate([ids, jnp.full((pad,), dump, jnp.int32)])


def _epad(e):
    q = NW * CHUNK
    return ((e + q - 1) // q) * q


def kernel(h1, h2, h3, h4,
           e_up_2_0, e_up_2_1, e_up_3_0, e_up_3_1, e_up_4_0, e_up_4_1,
           e_down_4_0, e_down_4_1, e_down_3_0, e_down_3_1,
           e_down_2_0, e_down_2_1,
           up2_W1, up2_b1, up2_W2, up2_b2,
           up3_W1, up3_b1, up3_W2, up3_b2,
           up4_W1, up4_b1, up4_W2, up4_b2,
           dn1_W1, dn1_b1, dn1_W2, dn1_b2,
           dn2_W1, dn2_b1, dn2_W2, dn2_b2,
           dn3_W1, dn3_b1, dn3_W2, dn3_b2):
    hs = {1: h1, 2: h2, 3: h3, 4: h4}
    up_e = {2: (e_up_2_0, e_up_2_1), 3: (e_up_3_0, e_up_3_1),
            4: (e_up_4_0, e_up_4_1)}
    dn_e = {4: (e_down_4_0, e_down_4_1), 3: (e_down_3_0, e_down_3_1),
            2: (e_down_2_0, e_down_2_1)}
    up_w = {2: (up2_W1, up2_b1, up2_W2, up2_b2),
            3: (up3_W1, up3_b1, up3_W2, up3_b2),
            4: (up4_W1, up4_b1, up4_W2, up4_b2)}
    dn_w = {1: (dn1_W1, dn1_b1, dn1_W2, dn1_b2),
            2: (dn2_W1, dn2_b1, dn2_W2, dn2_b2),
            3: (dn3_W1, dn3_b1, dn3_W2, dn3_b2)}

    for g in (2, 3, 4):
        e0, e1 = up_e[g]
        ng = hs[g].shape[0]
        ngp = _pad_rows(ng)
        e = e0.shape[1]
        ep = _epad(e)
        src = jnp.concatenate([_pad_edges(e0[0], 0, ep),
                               _pad_edges(e1[0], 0, ep)])
        dst = jnp.concatenate([_pad_edges(e0[1], ng, ep),
                               _pad_edges(e1[1] + ngp, ngp + ng, ep)])
        parts = _make_agg(hs[g - 1].shape[0], 2 * ngp, 2 * ep)(
            hs[g - 1], src, dst)
        hs[g] = _mlp_up(hs[g], parts, *up_w[g], ng, ngp)

    for g in (4, 3, 2):
        e0, e1 = dn_e[g]
        nd = hs[g - 1].shape[0]
        ndp = _pad_rows(nd)
        e = e0.shape[1]
        ep = _epad(e)
        src = jnp.concatenate([_pad_edges(e0[0], 0, ep),
                               _pad_edges(e1[0], 0, ep)])
        dst = jnp.concatenate([_pad_edges(e0[1], nd, ep),
                               _pad_edges(e1[1], nd, ep)])
        parts = _make_agg(hs[g].shape[0], ndp, 2 * ep)(hs[g], src, dst)
        hs[g - 1] = _mlp_dn(hs[g - 1], parts, *dn_w[g - 1], nd)

    return hs[1], hs[2], hs[3], hs[4]


# speedup vs baseline: 7.9702x; 1.0269x over previous
import functools

import jax
import jax.numpy as jnp
from jax import lax
from jax.experimental import pallas as pl
from jax.experimental.pallas import tpu as pltpu
from jax.experimental.pallas import tpu_sc as plsc

U = 32
NW = 32
ZC = 32
CHUNK = 256


MB = 1024
ND = 3
NI = 4


def _pad_rows(n):
    return ((n + MB - 1) // MB) * MB


@functools.cache
def _make_agg(nsrc, rp, etot):
    epw = etot // NW
    nch = epw // CHUNK
    rpt = rp // 16
    nz = rpt // ZC
    mesh = plsc.VectorSubcoreMesh(core_axis_name="c", subcore_axis_name="s")

    @functools.partial(
        pl.kernel,
        out_type=jax.ShapeDtypeStruct((2, rp, U), jnp.float32),
        mesh=mesh,
        compiler_params=pltpu.CompilerParams(use_tc_tiling_on_sc=False),
        scratch_types=[
            pltpu.VMEM_SHARED((rp, U), jnp.float32),
            pltpu.VMEM((ZC, U), jnp.float32),
            pltpu.VMEM((NI, CHUNK), jnp.int32),
            pltpu.VMEM((NI, CHUNK), jnp.int32),
            [pltpu.VMEM((CHUNK, U), jnp.float32)] * ND,
            [pltpu.SemaphoreType.DMA] * ND,
            [pltpu.SemaphoreType.DMA] * NI,
        ],
    )
    def agg(table, src, dst, out, acc, zbuf, src_v, dst_v, rows,
            gsem, ism):
        c = lax.axis_index("c")
        s = lax.axis_index("s")
        wid = c * 16 + s
        def zb(i, carry):
            zbuf[i, pl.ds(0, 16)] = jnp.zeros((16,), jnp.float32)
            zbuf[i, pl.ds(16, 16)] = jnp.zeros((16,), jnp.float32)
            return carry
        lax.fori_loop(0, ZC, zb, 0)
        base_r = s * rpt
        def zacc(k, carry):
            pltpu.sync_copy(zbuf, acc.at[pl.ds(base_r + k * ZC, ZC)])
            return carry
        lax.fori_loop(0, nz, zacc, 0)
        plsc.subcore_barrier()
        eb = wid * epw

        def load_idx(j):
            q = j % NI
            hs_ = pltpu.async_copy(src.at[pl.ds(eb + j * CHUNK, CHUNK)],
                                   src_v.at[q], ism[q])
            hd_ = pltpu.async_copy(dst.at[pl.ds(eb + j * CHUNK, CHUNK)],
                                   dst_v.at[q], ism[q])
            return hs_, hd_

        def gather(j):
            b = j % ND
            return pltpu.async_copy(table.at[src_v.at[j % NI]],
                                    rows[b], gsem[b])

        gh = [None] * nch
        ih = [None] * nch
        ih[0] = load_idx(0)
        if nch > 1:
            ih[1] = load_idx(1)
        ih[0][0].wait()
        gh[0] = gather(0)
        for j in range(nch):
            b = j % ND
            if j + 2 < nch:
                ih[j + 2] = load_idx(j + 2)
            if j + 1 < nch:
                ih[j + 1][0].wait()
                gh[j + 1] = gather(j + 1)
            gh[j].wait()
            ih[j][1].wait()
            pltpu.sync_copy(rows[b], acc.at[dst_v.at[j % NI]], add=True)
        plsc.subcore_barrier()
        pltpu.sync_copy(acc.at[pl.ds(base_r, rpt)],
                        out.at[c, pl.ds(base_r, rpt)])

    return agg


def _mlp_up(h, parts, w1, b1, w2, b2, n, ngp):
    def body(h_ref, p0_ref, p1_ref, w1_ref, b1_ref, w2_ref, b2_ref, o_ref):
        hv = h_ref[...]
        a0 = p0_ref[0] + p0_ref[1]
        a1 = p1_ref[0] + p1_ref[1]
        x1 = (jnp.dot(hv, w1_ref[0:U, :], preferred_element_type=jnp.float32)
              + jnp.dot(a0, w1_ref[U:2 * U, :], preferred_element_type=jnp.float32)
              + jnp.dot(a1, w1_ref[2 * U:3 * U, :], preferred_element_type=jnp.float32)
              + b1_ref[...][None, :])
        t = jnp.tanh(x1)
        o_ref[...] = (jnp.dot(t, w2_ref[...], preferred_element_type=jnp.float32)
                      + b2_ref[...][None, :])
    off = ngp // MB
    return pl.pallas_call(
        body,
        grid=(pl.cdiv(n, MB),),
        in_specs=[
            pl.BlockSpec((MB, U), lambda i: (i, 0)),
            pl.BlockSpec((2, MB, U), lambda i: (0, i, 0)),
            pl.BlockSpec((2, MB, U), lambda i: (0, off + i, 0)),
            pl.BlockSpec((3 * U, U), lambda i: (0, 0)),
            pl.BlockSpec((U,), lambda i: (0,)),
            pl.BlockSpec((U, U), lambda i: (0, 0)),
            pl.BlockSpec((U,), lambda i: (0,)),
        ],
        out_specs=pl.BlockSpec((MB, U), lambda i: (i, 0)),
        out_shape=jax.ShapeDtypeStruct((n, U), jnp.float32),
    )(h, parts, parts, w1, b1, w2, b2)


def _mlp_dn(h, parts, w1, b1, w2, b2, n):
    def body(h_ref, p_ref, w1_ref, b1_ref, w2_ref, b2_ref, o_ref):
        hv = h_ref[...]
        hd = p_ref[0] + p_ref[1]
        x1 = (jnp.dot(hv, w1_ref[0:U, :], preferred_element_type=jnp.float32)
              + jnp.dot(hd, w1_ref[U:2 * U, :], preferred_element_type=jnp.float32)
              + b1_ref[...][None, :])
        t = jnp.tanh(x1)
        o_ref[...] = (jnp.dot(t, w2_ref[...], preferred_element_type=jnp.float32)
                      + b2_ref[...][None, :])
    return pl.pallas_call(
        body,
        grid=(pl.cdiv(n, MB),),
        in_specs=[
            pl.BlockSpec((MB, U), lambda i: (i, 0)),
            pl.BlockSpec((2, MB, U), lambda i: (0, i, 0)),
            pl.BlockSpec((2 * U, U), lambda i: (0, 0)),
            pl.BlockSpec((U,), lambda i: (0,)),
            pl.BlockSpec((U, U), lambda i: (0, 0)),
            pl.BlockSpec((U,), lambda i: (0,)),
        ],
        out_specs=pl.BlockSpec((MB, U), lambda i: (i, 0)),
        out_shape=jax.ShapeDtypeStruct((n, U), jnp.float32),
    )(h, parts, w1, b1, w2, b2)


def _pad_edges(ids, dump, epad):
    pad = epad - ids.shape[0]
    if pad == 0:
        return ids
    return jnp.concatenate([ids, jnp.full((pad,), dump, jnp.int32)])


def _epad(e):
    q = NW * CHUNK
    return ((e + q - 1) // q) * q


def kernel(h1, h2, h3, h4,
           e_up_2_0, e_up_2_1, e_up_3_0, e_up_3_1, e_up_4_0, e_up_4_1,
           e_down_4_0, e_down_4_1, e_down_3_0, e_down_3_1,
           e_down_2_0, e_down_2_1,
           up2_W1, up2_b1, up2_W2, up2_b2,
           up3_W1, up3_b1, up3_W2, up3_b2,
           up4_W1, up4_b1, up4_W2, up4_b2,
           dn1_W1, dn1_b1, dn1_W2, dn1_b2,
           dn2_W1, dn2_b1, dn2_W2, dn2_b2,
           dn3_W1, dn3_b1, dn3_W2, dn3_b2):
    hs = {1: h1, 2: h2, 3: h3, 4: h4}
    up_e = {2: (e_up_2_0, e_up_2_1), 3: (e_up_3_0, e_up_3_1),
            4: (e_up_4_0, e_up_4_1)}
    dn_e = {4: (e_down_4_0, e_down_4_1), 3: (e_down_3_0, e_down_3_1),
            2: (e_down_2_0, e_down_2_1)}
    up_w = {2: (up2_W1, up2_b1, up2_W2, up2_b2),
            3: (up3_W1, up3_b1, up3_W2, up3_b2),
            4: (up4_W1, up4_b1, up4_W2, up4_b2)}
    dn_w = {1: (dn1_W1, dn1_b1, dn1_W2, dn1_b2),
            2: (dn2_W1, dn2_b1, dn2_W2, dn2_b2),
            3: (dn3_W1, dn3_b1, dn3_W2, dn3_b2)}

    for g in (2, 3, 4):
        e0, e1 = up_e[g]
        ng = hs[g].shape[0]
        ngp = _pad_rows(ng)
        e = e0.shape[1]
        ep = _epad(e)
        src = jnp.concatenate([_pad_edges(e0[0], 0, ep),
                               _pad_edges(e1[0], 0, ep)])
        dst = jnp.concatenate([_pad_edges(e0[1], ng, ep),
                               _pad_edges(e1[1] + ngp, ngp + ng, ep)])
        parts = _make_agg(hs[g - 1].shape[0], 2 * ngp, 2 * ep)(
            hs[g - 1], src, dst)
        hs[g] = _mlp_up(hs[g], parts, *up_w[g], ng, ngp)

    for g in (4, 3, 2):
        e0, e1 = dn_e[g]
        nd = hs[g - 1].shape[0]
        ndp = _pad_rows(nd)
        e = e0.shape[1]
        ep = _epad(e)
        src = jnp.concatenate([_pad_edges(e0[0], 0, ep),
                               _pad_edges(e1[0], 0, ep)])
        dst = jnp.concatenate([_pad_edges(e0[1], nd, ep),
                               _pad_edges(e1[1], nd, ep)])
        parts = _make_agg(hs[g].shape[0], ndp, 2 * ep)(hs[g], src, dst)
        hs[g - 1] = _mlp_dn(hs[g - 1], parts, *dn_w[g - 1], nd)

    return hs[1], hs[2], hs[3], hs[4]


# speedup vs baseline: 8.0864x; 1.0146x over previous
import functools

import jax
import jax.numpy as jnp
from jax import lax
from jax.experimental import pallas as pl
from jax.experimental.pallas import tpu as pltpu
from jax.experimental.pallas import tpu_sc as plsc

U = 32
NW = 32
ZC = 32
CHUNK = 256


MB = 1024
ND = 3
NI = 4


def _pad_rows(n):
    return ((n + MB - 1) // MB) * MB


@functools.cache
def _make_agg(nsrc, rp, etot):
    epw = etot // NW
    nch = epw // CHUNK
    rpt = rp // 16
    nz = rpt // ZC
    mesh = plsc.VectorSubcoreMesh(core_axis_name="c", subcore_axis_name="s")

    @functools.partial(
        pl.kernel,
        out_type=jax.ShapeDtypeStruct((2, rp, U), jnp.float32),
        mesh=mesh,
        compiler_params=pltpu.CompilerParams(use_tc_tiling_on_sc=False),
        scratch_types=[
            pltpu.VMEM_SHARED((rp, U), jnp.float32),
            pltpu.VMEM((ZC, U), jnp.float32),
            pltpu.VMEM((NI, CHUNK), jnp.int32),
            pltpu.VMEM((NI, CHUNK), jnp.int32),
            [pltpu.VMEM((CHUNK, U), jnp.float32)] * ND,
            [pltpu.SemaphoreType.DMA] * ND,
            [pltpu.SemaphoreType.DMA] * ND,
            [pltpu.SemaphoreType.DMA] * NI,
        ],
    )
    def agg(table, src, dst, out, acc, zbuf, src_v, dst_v, rows,
            gsem, ssem, ism):
        c = lax.axis_index("c")
        s = lax.axis_index("s")
        wid = c * 16 + s
        def zb(i, carry):
            zbuf[i, pl.ds(0, 16)] = jnp.zeros((16,), jnp.float32)
            zbuf[i, pl.ds(16, 16)] = jnp.zeros((16,), jnp.float32)
            return carry
        lax.fori_loop(0, ZC, zb, 0)
        base_r = s * rpt
        def zacc(k, carry):
            pltpu.sync_copy(zbuf, acc.at[pl.ds(base_r + k * ZC, ZC)])
            return carry
        lax.fori_loop(0, nz, zacc, 0)
        plsc.subcore_barrier()
        eb = wid * epw

        def load_idx(j):
            q = j % NI
            hs_ = pltpu.async_copy(src.at[pl.ds(eb + j * CHUNK, CHUNK)],
                                   src_v.at[q], ism[q])
            hd_ = pltpu.async_copy(dst.at[pl.ds(eb + j * CHUNK, CHUNK)],
                                   dst_v.at[q], ism[q])
            return hs_, hd_

        def gather(j):
            b = j % ND
            return pltpu.async_copy(table.at[src_v.at[j % NI]],
                                    rows[b], gsem[b])

        gh = [None] * nch
        ih = [None] * nch
        sh = [None] * nch
        ih[0] = load_idx(0)
        if nch > 1:
            ih[1] = load_idx(1)
        ih[0][0].wait()
        gh[0] = gather(0)
        for j in range(nch):
            b = j % ND
            if j >= 2:
                sh[j - 2].wait()
            if j + 2 < nch:
                ih[j + 2] = load_idx(j + 2)
            if j + 1 < nch:
                ih[j + 1][0].wait()
                gh[j + 1] = gather(j + 1)
            gh[j].wait()
            ih[j][1].wait()
            sh[j] = pltpu.async_copy(rows[b], acc.at[dst_v.at[j % NI]],
                                     ssem[b], add=True)
        if nch >= 2:
            sh[nch - 2].wait()
        sh[nch - 1].wait()
        plsc.subcore_barrier()
        pltpu.sync_copy(acc.at[pl.ds(base_r, rpt)],
                        out.at[c, pl.ds(base_r, rpt)])

    return agg


def _mlp_up(h, parts, w1, b1, w2, b2, n, ngp):
    def body(h_ref, p0_ref, p1_ref, w1_ref, b1_ref, w2_ref, b2_ref, o_ref):
        hv = h_ref[...]
        a0 = p0_ref[0] + p0_ref[1]
        a1 = p1_ref[0] + p1_ref[1]
        x1 = (jnp.dot(hv, w1_ref[0:U, :], preferred_element_type=jnp.float32)
              + jnp.dot(a0, w1_ref[U:2 * U, :], preferred_element_type=jnp.float32)
              + jnp.dot(a1, w1_ref[2 * U:3 * U, :], preferred_element_type=jnp.float32)
              + b1_ref[...][None, :])
        t = jnp.tanh(x1)
        o_ref[...] = (jnp.dot(t, w2_ref[...], preferred_element_type=jnp.float32)
                      + b2_ref[...][None, :])
    off = ngp // MB
    return pl.pallas_call(
        body,
        grid=(pl.cdiv(n, MB),),
        in_specs=[
            pl.BlockSpec((MB, U), lambda i: (i, 0)),
            pl.BlockSpec((2, MB, U), lambda i: (0, i, 0)),
            pl.BlockSpec((2, MB, U), lambda i: (0, off + i, 0)),
            pl.BlockSpec((3 * U, U), lambda i: (0, 0)),
            pl.BlockSpec((U,), lambda i: (0,)),
            pl.BlockSpec((U, U), lambda i: (0, 0)),
            pl.BlockSpec((U,), lambda i: (0,)),
        ],
        out_specs=pl.BlockSpec((MB, U), lambda i: (i, 0)),
        out_shape=jax.ShapeDtypeStruct((n, U), jnp.float32),
    )(h, parts, parts, w1, b1, w2, b2)


def _mlp_dn(h, parts, w1, b1, w2, b2, n):
    def body(h_ref, p_ref, w1_ref, b1_ref, w2_ref, b2_ref, o_ref):
        hv = h_ref[...]
        hd = p_ref[0] + p_ref[1]
        x1 = (jnp.dot(hv, w1_ref[0:U, :], preferred_element_type=jnp.float32)
              + jnp.dot(hd, w1_ref[U:2 * U, :], preferred_element_type=jnp.float32)
              + b1_ref[...][None, :])
        t = jnp.tanh(x1)
        o_ref[...] = (jnp.dot(t, w2_ref[...], preferred_element_type=jnp.float32)
                      + b2_ref[...][None, :])
    return pl.pallas_call(
        body,
        grid=(pl.cdiv(n, MB),),
        in_specs=[
            pl.BlockSpec((MB, U), lambda i: (i, 0)),
            pl.BlockSpec((2, MB, U), lambda i: (0, i, 0)),
            pl.BlockSpec((2 * U, U), lambda i: (0, 0)),
            pl.BlockSpec((U,), lambda i: (0,)),
            pl.BlockSpec((U, U), lambda i: (0, 0)),
            pl.BlockSpec((U,), lambda i: (0,)),
        ],
        out_specs=pl.BlockSpec((MB, U), lambda i: (i, 0)),
        out_shape=jax.ShapeDtypeStruct((n, U), jnp.float32),
    )(h, parts, w1, b1, w2, b2)


def _pad_edges(ids, dump, epad):
    pad = epad - ids.shape[0]
    if pad == 0:
        return ids
    return jnp.concatenate([ids, jnp.full((pad,), dump, jnp.int32)])


def _epad(e):
    q = NW * CHUNK
    return ((e + q - 1) // q) * q


def kernel(h1, h2, h3, h4,
           e_up_2_0, e_up_2_1, e_up_3_0, e_up_3_1, e_up_4_0, e_up_4_1,
           e_down_4_0, e_down_4_1, e_down_3_0, e_down_3_1,
           e_down_2_0, e_down_2_1,
           up2_W1, up2_b1, up2_W2, up2_b2,
           up3_W1, up3_b1, up3_W2, up3_b2,
           up4_W1, up4_b1, up4_W2, up4_b2,
           dn1_W1, dn1_b1, dn1_W2, dn1_b2,
           dn2_W1, dn2_b1, dn2_W2, dn2_b2,
           dn3_W1, dn3_b1, dn3_W2, dn3_b2):
    hs = {1: h1, 2: h2, 3: h3, 4: h4}
    up_e = {2: (e_up_2_0, e_up_2_1), 3: (e_up_3_0, e_up_3_1),
            4: (e_up_4_0, e_up_4_1)}
    dn_e = {4: (e_down_4_0, e_down_4_1), 3: (e_down_3_0, e_down_3_1),
            2: (e_down_2_0, e_down_2_1)}
    up_w = {2: (up2_W1, up2_b1, up2_W2, up2_b2),
            3: (up3_W1, up3_b1, up3_W2, up3_b2),
            4: (up4_W1, up4_b1, up4_W2, up4_b2)}
    dn_w = {1: (dn1_W1, dn1_b1, dn1_W2, dn1_b2),
            2: (dn2_W1, dn2_b1, dn2_W2, dn2_b2),
            3: (dn3_W1, dn3_b1, dn3_W2, dn3_b2)}

    for g in (2, 3, 4):
        e0, e1 = up_e[g]
        ng = hs[g].shape[0]
        ngp = _pad_rows(ng)
        e = e0.shape[1]
        ep = _epad(e)
        src = jnp.concatenate([_pad_edges(e0[0], 0, ep),
                               _pad_edges(e1[0], 0, ep)])
        dst = jnp.concatenate([_pad_edges(e0[1], ng, ep),
                               _pad_edges(e1[1] + ngp, ngp + ng, ep)])
        parts = _make_agg(hs[g - 1].shape[0], 2 * ngp, 2 * ep)(
            hs[g - 1], src, dst)
        hs[g] = _mlp_up(hs[g], parts, *up_w[g], ng, ngp)

    for g in (4, 3, 2):
        e0, e1 = dn_e[g]
        nd = hs[g - 1].shape[0]
        ndp = _pad_rows(nd)
        e = e0.shape[1]
        ep = _epad(e)
        src = jnp.concatenate([_pad_edges(e0[0], 0, ep),
                               _pad_edges(e1[0], 0, ep)])
        dst = jnp.concatenate([_pad_edges(e0[1], nd, ep),
                               _pad_edges(e1[1], nd, ep)])
        parts = _make_agg(hs[g].shape[0], ndp, 2 * ep)(hs[g], src, dst)
        hs[g - 1] = _mlp_dn(hs[g - 1], parts, *dn_w[g - 1], nd)

    return hs[1], hs[2], hs[3], hs[4]


# speedup vs baseline: 8.2901x; 1.0252x over previous
import functools

import jax
import jax.numpy as jnp
from jax import lax
from jax.experimental import pallas as pl
from jax.experimental.pallas import tpu as pltpu
from jax.experimental.pallas import tpu_sc as plsc

U = 32
NW = 32
ZC = 32
CHUNK = 256


MB = 1024
ND = 3
NI = 4


def _pad_rows(n):
    return ((n + MB - 1) // MB) * MB


@functools.cache
def _make_agg(nsrc, rp, etot):
    epw = etot // NW
    chunk = 512
    while rp * U + 16 * (ZC * U + 2 * NI * chunk + ND * chunk * U) > 2095000:
        chunk //= 2
    nch = epw // chunk
    rpt = rp // 16
    nz = rpt // ZC
    mesh = plsc.VectorSubcoreMesh(core_axis_name="c", subcore_axis_name="s")

    @functools.partial(
        pl.kernel,
        out_type=jax.ShapeDtypeStruct((2, rp, U), jnp.float32),
        mesh=mesh,
        compiler_params=pltpu.CompilerParams(use_tc_tiling_on_sc=False),
        scratch_types=[
            pltpu.VMEM_SHARED((rp, U), jnp.float32),
            pltpu.VMEM((ZC, U), jnp.float32),
            pltpu.VMEM((NI, chunk), jnp.int32),
            pltpu.VMEM((NI, chunk), jnp.int32),
            [pltpu.VMEM((chunk, U), jnp.float32)] * ND,
            [pltpu.SemaphoreType.DMA] * ND,
            [pltpu.SemaphoreType.DMA] * ND,
            [pltpu.SemaphoreType.DMA] * NI,
            pltpu.SemaphoreType.DMA,
        ],
    )
    def agg(table, src, dst, out, acc, zbuf, src_v, dst_v, rows,
            gsem, ssem, ism, zsem):
        c = lax.axis_index("c")
        s = lax.axis_index("s")
        wid = c * 16 + s
        eb = wid * epw
        base_r = s * rpt

        def load_idx(j):
            q = j % NI
            hs_ = pltpu.async_copy(src.at[pl.ds(eb + j * chunk, chunk)],
                                   src_v.at[q], ism[q])
            hd_ = pltpu.async_copy(dst.at[pl.ds(eb + j * chunk, chunk)],
                                   dst_v.at[q], ism[q])
            return hs_, hd_

        def gather(j):
            b = j % ND
            return pltpu.async_copy(table.at[src_v.at[j % NI]],
                                    rows[b], gsem[b])

        ih = [None] * nch
        ih[0] = load_idx(0)
        if nch > 1:
            ih[1] = load_idx(1)
        def zb(i, carry):
            zbuf[i, pl.ds(0, 16)] = jnp.zeros((16,), jnp.float32)
            zbuf[i, pl.ds(16, 16)] = jnp.zeros((16,), jnp.float32)
            return carry
        lax.fori_loop(0, ZC, zb, 0)
        def zacc(k, carry):
            pltpu.async_copy(zbuf, acc.at[pl.ds(base_r + k * ZC, ZC)], zsem)
            return carry
        lax.fori_loop(0, nz, zacc, 0)
        gh = [None] * nch
        sh = [None] * nch
        ih[0][0].wait()
        gh[0] = gather(0)
        pltpu.make_async_copy(out.at[c, pl.ds(base_r, rpt)],
                              acc.at[pl.ds(base_r, rpt)], zsem).wait()
        plsc.subcore_barrier()
        for j in range(nch):
            b = j % ND
            if j >= 2:
                sh[j - 2].wait()
            if j + 2 < nch:
                ih[j + 2] = load_idx(j + 2)
            if j + 1 < nch:
                ih[j + 1][0].wait()
                gh[j + 1] = gather(j + 1)
            gh[j].wait()
            ih[j][1].wait()
            sh[j] = pltpu.async_copy(rows[b], acc.at[dst_v.at[j % NI]],
                                     ssem[b], add=True)
        if nch >= 2:
            sh[nch - 2].wait()
        sh[nch - 1].wait()
        plsc.subcore_barrier()
        pltpu.sync_copy(acc.at[pl.ds(base_r, rpt)],
                        out.at[c, pl.ds(base_r, rpt)])

    return agg


def _mlp_up(h, parts, w1, b1, w2, b2, n, ngp):
    def body(h_ref, p0_ref, p1_ref, w1_ref, b1_ref, w2_ref, b2_ref, o_ref):
        hv = h_ref[...]
        a0 = p0_ref[0] + p0_ref[1]
        a1 = p1_ref[0] + p1_ref[1]
        x1 = (jnp.dot(hv, w1_ref[0:U, :], preferred_element_type=jnp.float32)
              + jnp.dot(a0, w1_ref[U:2 * U, :], preferred_element_type=jnp.float32)
              + jnp.dot(a1, w1_ref[2 * U:3 * U, :], preferred_element_type=jnp.float32)
              + b1_ref[...][None, :])
        t = jnp.tanh(x1)
        o_ref[...] = (jnp.dot(t, w2_ref[...], preferred_element_type=jnp.float32)
                      + b2_ref[...][None, :])
    off = ngp // MB
    return pl.pallas_call(
        body,
        grid=(pl.cdiv(n, MB),),
        in_specs=[
            pl.BlockSpec((MB, U), lambda i: (i, 0)),
            pl.BlockSpec((2, MB, U), lambda i: (0, i, 0)),
            pl.BlockSpec((2, MB, U), lambda i: (0, off + i, 0)),
            pl.BlockSpec((3 * U, U), lambda i: (0, 0)),
            pl.BlockSpec((U,), lambda i: (0,)),
            pl.BlockSpec((U, U), lambda i: (0, 0)),
            pl.BlockSpec((U,), lambda i: (0,)),
        ],
        out_specs=pl.BlockSpec((MB, U), lambda i: (i, 0)),
        out_shape=jax.ShapeDtypeStruct((n, U), jnp.float32),
    )(h, parts, parts, w1, b1, w2, b2)


def _mlp_dn(h, parts, w1, b1, w2, b2, n):
    def body(h_ref, p_ref, w1_ref, b1_ref, w2_ref, b2_ref, o_ref):
        hv = h_ref[...]
        hd = p_ref[0] + p_ref[1]
        x1 = (jnp.dot(hv, w1_ref[0:U, :], preferred_element_type=jnp.float32)
              + jnp.dot(hd, w1_ref[U:2 * U, :], preferred_element_type=jnp.float32)
              + b1_ref[...][None, :])
        t = jnp.tanh(x1)
        o_ref[...] = (jnp.dot(t, w2_ref[...], preferred_element_type=jnp.float32)
                      + b2_ref[...][None, :])
    return pl.pallas_call(
        body,
        grid=(pl.cdiv(n, MB),),
        in_specs=[
            pl.BlockSpec((MB, U), lambda i: (i, 0)),
            pl.BlockSpec((2, MB, U), lambda i: (0, i, 0)),
            pl.BlockSpec((2 * U, U), lambda i: (0, 0)),
            pl.BlockSpec((U,), lambda i: (0,)),
            pl.BlockSpec((U, U), lambda i: (0, 0)),
            pl.BlockSpec((U,), lambda i: (0,)),
        ],
        out_specs=pl.BlockSpec((MB, U), lambda i: (i, 0)),
        out_shape=jax.ShapeDtypeStruct((n, U), jnp.float32),
    )(h, parts, w1, b1, w2, b2)


def _pad_edges(ids, dump, epad):
    pad = epad - ids.shape[0]
    if pad == 0:
        return ids
    return jnp.concatenate([ids, jnp.full((pad,), dump, jnp.int32)])


def _epad(e):
    q = NW * CHUNK
    return ((e + q - 1) // q) * q


def kernel(h1, h2, h3, h4,
           e_up_2_0, e_up_2_1, e_up_3_0, e_up_3_1, e_up_4_0, e_up_4_1,
           e_down_4_0, e_down_4_1, e_down_3_0, e_down_3_1,
           e_down_2_0, e_down_2_1,
           up2_W1, up2_b1, up2_W2, up2_b2,
           up3_W1, up3_b1, up3_W2, up3_b2,
           up4_W1, up4_b1, up4_W2, up4_b2,
           dn1_W1, dn1_b1, dn1_W2, dn1_b2,
           dn2_W1, dn2_b1, dn2_W2, dn2_b2,
           dn3_W1, dn3_b1, dn3_W2, dn3_b2):
    hs = {1: h1, 2: h2, 3: h3, 4: h4}
    up_e = {2: (e_up_2_0, e_up_2_1), 3: (e_up_3_0, e_up_3_1),
            4: (e_up_4_0, e_up_4_1)}
    dn_e = {4: (e_down_4_0, e_down_4_1), 3: (e_down_3_0, e_down_3_1),
            2: (e_down_2_0, e_down_2_1)}
    up_w = {2: (up2_W1, up2_b1, up2_W2, up2_b2),
            3: (up3_W1, up3_b1, up3_W2, up3_b2),
            4: (up4_W1, up4_b1, up4_W2, up4_b2)}
    dn_w = {1: (dn1_W1, dn1_b1, dn1_W2, dn1_b2),
            2: (dn2_W1, dn2_b1, dn2_W2, dn2_b2),
            3: (dn3_W1, dn3_b1, dn3_W2, dn3_b2)}

    for g in (2, 3, 4):
        e0, e1 = up_e[g]
        ng = hs[g].shape[0]
        ngp = _pad_rows(ng)
        e = e0.shape[1]
        ep = _epad(e)
        src = jnp.concatenate([_pad_edges(e0[0], 0, ep),
                               _pad_edges(e1[0], 0, ep)])
        dst = jnp.concatenate([_pad_edges(e0[1], ng, ep),
                               _pad_edges(e1[1] + ngp, ngp + ng, ep)])
        parts = _make_agg(hs[g - 1].shape[0], 2 * ngp, 2 * ep)(
            hs[g - 1], src, dst)
        hs[g] = _mlp_up(hs[g], parts, *up_w[g], ng, ngp)

    for g in (4, 3, 2):
        e0, e1 = dn_e[g]
        nd = hs[g - 1].shape[0]
        ndp = _pad_rows(nd)
        e = e0.shape[1]
        ep = _epad(e)
        src = jnp.concatenate([_pad_edges(e0[0], 0, ep),
                               _pad_edges(e1[0], 0, ep)])
        dst = jnp.concatenate([_pad_edges(e0[1], nd, ep),
                               _pad_edges(e1[1], nd, ep)])
        parts = _make_agg(hs[g].shape[0], ndp, 2 * ep)(hs[g], src, dst)
        hs[g - 1] = _mlp_dn(hs[g - 1], parts, *dn_w[g - 1], nd)

    return hs[1], hs[2], hs[3], hs[4]


# speedup vs baseline: 8.2923x; 1.0003x over previous
import functools

import jax
import jax.numpy as jnp
from jax import lax
from jax.experimental import pallas as pl
from jax.experimental.pallas import tpu as pltpu
from jax.experimental.pallas import tpu_sc as plsc

U = 32
NW = 32
ZC = 32
CHUNK = 256


MB = 1024
ND = 3
NI = 4


def _pad_rows(n):
    return ((n + MB - 1) // MB) * MB


@functools.cache
def _make_agg(nsrc, rp, etot):
    epw = etot // NW
    chunk = 512
    while rp * U + 16 * (ZC * U + 2 * NI * chunk + ND * chunk * U) > 2095000:
        chunk //= 2
    nch = epw // chunk
    rpt = rp // 16
    nz = rpt // ZC
    mesh = plsc.VectorSubcoreMesh(core_axis_name="c", subcore_axis_name="s")

    @functools.partial(
        pl.kernel,
        out_type=jax.ShapeDtypeStruct((2, rp, U), jnp.float32),
        mesh=mesh,
        compiler_params=pltpu.CompilerParams(use_tc_tiling_on_sc=False),
        scratch_types=[
            pltpu.VMEM_SHARED((rp, U), jnp.float32),
            pltpu.VMEM((ZC, U), jnp.float32),
            pltpu.VMEM((NI, chunk), jnp.int32),
            pltpu.VMEM((NI, chunk), jnp.int32),
            [pltpu.VMEM((chunk, U), jnp.float32)] * ND,
            [pltpu.SemaphoreType.DMA] * ND,
            [pltpu.SemaphoreType.DMA] * ND,
            [pltpu.SemaphoreType.DMA] * NI,
            pltpu.SemaphoreType.DMA,
        ],
    )
    def agg(table, src, dst, out, acc, zbuf, src_v, dst_v, rows,
            gsem, ssem, ism, zsem):
        c = lax.axis_index("c")
        s = lax.axis_index("s")
        wid = c * 16 + s
        eb = wid * epw
        base_r = s * rpt

        def load_idx(j):
            q = j % NI
            hs_ = pltpu.async_copy(src.at[pl.ds(eb + j * chunk, chunk)],
                                   src_v.at[q], ism[q])
            hd_ = pltpu.async_copy(dst.at[pl.ds(eb + j * chunk, chunk)],
                                   dst_v.at[q], ism[q])
            return hs_, hd_

        def gather(j):
            b = j % ND
            return pltpu.async_copy(table.at[src_v.at[j % NI]],
                                    rows[b], gsem[b])

        ih = [None] * nch
        ih[0] = load_idx(0)
        if nch > 1:
            ih[1] = load_idx(1)
        def zb(i, carry):
            zbuf[i, pl.ds(0, 16)] = jnp.zeros((16,), jnp.float32)
            zbuf[i, pl.ds(16, 16)] = jnp.zeros((16,), jnp.float32)
            return carry
        lax.fori_loop(0, ZC, zb, 0)
        def zacc(k, carry):
            pltpu.async_copy(zbuf, acc.at[pl.ds(base_r + k * ZC, ZC)], zsem)
            return carry
        lax.fori_loop(0, nz, zacc, 0)
        gh = [None] * nch
        sh = [None] * nch
        ih[0][0].wait()
        gh[0] = gather(0)
        pltpu.make_async_copy(out.at[c, pl.ds(base_r, rpt)],
                              acc.at[pl.ds(base_r, rpt)], zsem).wait()
        plsc.subcore_barrier()
        for j in range(nch):
            b = j % ND
            if j >= 2:
                sh[j - 2].wait()
            if j + 2 < nch:
                ih[j + 2] = load_idx(j + 2)
            if j + 1 < nch:
                ih[j + 1][0].wait()
                gh[j + 1] = gather(j + 1)
            gh[j].wait()
            ih[j][1].wait()
            sh[j] = pltpu.async_copy(rows[b], acc.at[dst_v.at[j % NI]],
                                     ssem[b], add=True)
        if nch >= 2:
            sh[nch - 2].wait()
        sh[nch - 1].wait()
        plsc.subcore_barrier()
        pltpu.sync_copy(acc.at[pl.ds(base_r, rpt)],
                        out.at[c, pl.ds(base_r, rpt)])

    return agg


def _mlp_up(h, parts, w1, b1, w2, b2, n, ngp):
    def body(h_ref, p0_ref, p1_ref, w1_ref, b1_ref, w2_ref, b2_ref, o_ref):
        hv = h_ref[...]
        a0 = p0_ref[0] + p0_ref[1]
        a1 = p1_ref[0] + p1_ref[1]
        x = jnp.concatenate([hv, a0, a1], axis=-1)
        x1 = (jnp.dot(x, w1_ref[...], preferred_element_type=jnp.float32)
              + b1_ref[...][None, :])
        t = jnp.tanh(x1)
        o_ref[...] = (jnp.dot(t, w2_ref[...], preferred_element_type=jnp.float32)
                      + b2_ref[...][None, :])
    off = ngp // MB
    return pl.pallas_call(
        body,
        grid=(pl.cdiv(n, MB),),
        in_specs=[
            pl.BlockSpec((MB, U), lambda i: (i, 0)),
            pl.BlockSpec((2, MB, U), lambda i: (0, i, 0)),
            pl.BlockSpec((2, MB, U), lambda i: (0, off + i, 0)),
            pl.BlockSpec((3 * U, U), lambda i: (0, 0)),
            pl.BlockSpec((U,), lambda i: (0,)),
            pl.BlockSpec((U, U), lambda i: (0, 0)),
            pl.BlockSpec((U,), lambda i: (0,)),
        ],
        out_specs=pl.BlockSpec((MB, U), lambda i: (i, 0)),
        out_shape=jax.ShapeDtypeStruct((n, U), jnp.float32),
    )(h, parts, parts, w1, b1, w2, b2)


def _mlp_dn(h, parts, w1, b1, w2, b2, n):
    def body(h_ref, p_ref, w1_ref, b1_ref, w2_ref, b2_ref, o_ref):
        hv = h_ref[...]
        hd = p_ref[0] + p_ref[1]
        x = jnp.concatenate([hv, hd], axis=-1)
        x1 = (jnp.dot(x, w1_ref[...], preferred_element_type=jnp.float32)
              + b1_ref[...][None, :])
        t = jnp.tanh(x1)
        o_ref[...] = (jnp.dot(t, w2_ref[...], preferred_element_type=jnp.float32)
                      + b2_ref[...][None, :])
    return pl.pallas_call(
        body,
        grid=(pl.cdiv(n, MB),),
        in_specs=[
            pl.BlockSpec((MB, U), lambda i: (i, 0)),
            pl.BlockSpec((2, MB, U), lambda i: (0, i, 0)),
            pl.BlockSpec((2 * U, U), lambda i: (0, 0)),
            pl.BlockSpec((U,), lambda i: (0,)),
            pl.BlockSpec((U, U), lambda i: (0, 0)),
            pl.BlockSpec((U,), lambda i: (0,)),
        ],
        out_specs=pl.BlockSpec((MB, U), lambda i: (i, 0)),
        out_shape=jax.ShapeDtypeStruct((n, U), jnp.float32),
    )(h, parts, w1, b1, w2, b2)


def _pad_edges(ids, dump, epad):
    pad = epad - ids.shape[0]
    if pad == 0:
        return ids
    return jnp.concatenate([ids, jnp.full((pad,), dump, jnp.int32)])


def _epad(e):
    q = NW * CHUNK
    return ((e + q - 1) // q) * q


def kernel(h1, h2, h3, h4,
           e_up_2_0, e_up_2_1, e_up_3_0, e_up_3_1, e_up_4_0, e_up_4_1,
           e_down_4_0, e_down_4_1, e_down_3_0, e_down_3_1,
           e_down_2_0, e_down_2_1,
           up2_W1, up2_b1, up2_W2, up2_b2,
           up3_W1, up3_b1, up3_W2, up3_b2,
           up4_W1, up4_b1, up4_W2, up4_b2,
           dn1_W1, dn1_b1, dn1_W2, dn1_b2,
           dn2_W1, dn2_b1, dn2_W2, dn2_b2,
           dn3_W1, dn3_b1, dn3_W2, dn3_b2):
    hs = {1: h1, 2: h2, 3: h3, 4: h4}
    up_e = {2: (e_up_2_0, e_up_2_1), 3: (e_up_3_0, e_up_3_1),
            4: (e_up_4_0, e_up_4_1)}
    dn_e = {4: (e_down_4_0, e_down_4_1), 3: (e_down_3_0, e_down_3_1),
            2: (e_down_2_0, e_down_2_1)}
    up_w = {2: (up2_W1, up2_b1, up2_W2, up2_b2),
            3: (up3_W1, up3_b1, up3_W2, up3_b2),
            4: (up4_W1, up4_b1, up4_W2, up4_b2)}
    dn_w = {1: (dn1_W1, dn1_b1, dn1_W2, dn1_b2),
            2: (dn2_W1, dn2_b1, dn2_W2, dn2_b2),
            3: (dn3_W1, dn3_b1, dn3_W2, dn3_b2)}

    for g in (2, 3, 4):
        e0, e1 = up_e[g]
        ng = hs[g].shape[0]
        ngp = _pad_rows(ng)
        e = e0.shape[1]
        ep = _epad(e)
        src = jnp.concatenate([_pad_edges(e0[0], 0, ep),
                               _pad_edges(e1[0], 0, ep)])
        dst = jnp.concatenate([_pad_edges(e0[1], ng, ep),
                               _pad_edges(e1[1] + ngp, ngp + ng, ep)])
        parts = _make_agg(hs[g - 1].shape[0], 2 * ngp, 2 * ep)(
            hs[g - 1], src, dst)
        hs[g] = _mlp_up(hs[g], parts, *up_w[g], ng, ngp)

    for g in (4, 3, 2):
        e0, e1 = dn_e[g]
        nd = hs[g - 1].shape[0]
        ndp = _pad_rows(nd)
        e = e0.shape[1]
        ep = _epad(e)
        src = jnp.concatenate([_pad_edges(e0[0], 0, ep),
                               _pad_edges(e1[0], 0, ep)])
        dst = jnp.concatenate([_pad_edges(e0[1], nd, ep),
                               _pad_edges(e1[1], nd, ep)])
        parts = _make_agg(hs[g].shape[0], ndp, 2 * ep)(hs[g], src, dst)
        hs[g - 1] = _mlp_dn(hs[g - 1], parts, *dn_w[g - 1], nd)

    return hs[1], hs[2], hs[3], hs[4]


# speedup vs baseline: 8.2935x; 1.0001x over previous
import functools

import jax
import jax.numpy as jnp
from jax import lax
from jax.experimental import pallas as pl
from jax.experimental.pallas import tpu as pltpu
from jax.experimental.pallas import tpu_sc as plsc

U = 32
NW = 32
ZC = 32
CHUNK = 256


MB = 1024
ND = 3
NI = 4


def _pad_rows(n):
    return ((n + MB - 1) // MB) * MB


@functools.cache
def _make_agg(nsrc, rp, etot):
    epw = etot // NW
    chunk = 512
    while rp * U + 16 * (ZC * U + 2 * NI * chunk + ND * chunk * U) > 2095000:
        chunk //= 2
    assert epw % chunk == 0
    nch = epw // chunk
    rpt = rp // 16
    nz = rpt // ZC
    mesh = plsc.VectorSubcoreMesh(core_axis_name="c", subcore_axis_name="s")

    @functools.partial(
        pl.kernel,
        out_type=jax.ShapeDtypeStruct((2, rp, U), jnp.float32),
        mesh=mesh,
        compiler_params=pltpu.CompilerParams(use_tc_tiling_on_sc=False),
        scratch_types=[
            pltpu.VMEM_SHARED((rp, U), jnp.float32),
            pltpu.VMEM((ZC, U), jnp.float32),
            pltpu.VMEM((NI, chunk), jnp.int32),
            pltpu.VMEM((NI, chunk), jnp.int32),
            [pltpu.VMEM((chunk, U), jnp.float32)] * ND,
            [pltpu.SemaphoreType.DMA] * ND,
            [pltpu.SemaphoreType.DMA] * ND,
            [pltpu.SemaphoreType.DMA] * NI,
            pltpu.SemaphoreType.DMA,
        ],
    )
    def agg(table, src, dst, out, acc, zbuf, src_v, dst_v, rows,
            gsem, ssem, ism, zsem):
        c = lax.axis_index("c")
        s = lax.axis_index("s")
        wid = c * 16 + s
        eb = wid * epw
        base_r = s * rpt

        def load_idx(j):
            q = j % NI
            hs_ = pltpu.async_copy(src.at[pl.ds(eb + j * chunk, chunk)],
                                   src_v.at[q], ism[q])
            hd_ = pltpu.async_copy(dst.at[pl.ds(eb + j * chunk, chunk)],
                                   dst_v.at[q], ism[q])
            return hs_, hd_

        def gather(j):
            b = j % ND
            return pltpu.async_copy(table.at[src_v.at[j % NI]],
                                    rows[b], gsem[b])

        ih = [None] * nch
        ih[0] = load_idx(0)
        if nch > 1:
            ih[1] = load_idx(1)
        def zb(i, carry):
            zbuf[i, pl.ds(0, 16)] = jnp.zeros((16,), jnp.float32)
            zbuf[i, pl.ds(16, 16)] = jnp.zeros((16,), jnp.float32)
            return carry
        lax.fori_loop(0, ZC, zb, 0)
        def zacc(k, carry):
            pltpu.async_copy(zbuf, acc.at[pl.ds(base_r + k * ZC, ZC)], zsem)
            return carry
        lax.fori_loop(0, nz, zacc, 0)
        gh = [None] * nch
        sh = [None] * nch
        ih[0][0].wait()
        gh[0] = gather(0)
        pltpu.make_async_copy(out.at[c, pl.ds(base_r, rpt)],
                              acc.at[pl.ds(base_r, rpt)], zsem).wait()
        plsc.subcore_barrier()
        for j in range(nch):
            b = j % ND
            if j >= 2:
                sh[j - 2].wait()
            if j + 2 < nch:
                ih[j + 2] = load_idx(j + 2)
            if j + 1 < nch:
                ih[j + 1][0].wait()
                gh[j + 1] = gather(j + 1)
            gh[j].wait()
            ih[j][1].wait()
            sh[j] = pltpu.async_copy(rows[b], acc.at[dst_v.at[j % NI]],
                                     ssem[b], add=True)
        if nch >= 2:
            sh[nch - 2].wait()
        sh[nch - 1].wait()
        plsc.subcore_barrier()
        pltpu.sync_copy(acc.at[pl.ds(base_r, rpt)],
                        out.at[c, pl.ds(base_r, rpt)])

    return agg


def _mlp_up(h, parts, w1, b1, w2, b2, n, ngp):
    def body(h_ref, p0_ref, p1_ref, w1_ref, b1_ref, w2_ref, b2_ref, o_ref):
        hv = h_ref[...]
        a0 = p0_ref[0] + p0_ref[1]
        a1 = p1_ref[0] + p1_ref[1]
        x = jnp.concatenate([hv, a0, a1], axis=-1)
        x1 = (jnp.dot(x, w1_ref[...], preferred_element_type=jnp.float32)
              + b1_ref[...][None, :])
        t = jnp.tanh(x1)
        o_ref[...] = (jnp.dot(t, w2_ref[...], preferred_element_type=jnp.float32)
                      + b2_ref[...][None, :])
    off = ngp // MB
    return pl.pallas_call(
        body,
        grid=(pl.cdiv(n, MB),),
        in_specs=[
            pl.BlockSpec((MB, U), lambda i: (i, 0)),
            pl.BlockSpec((2, MB, U), lambda i: (0, i, 0)),
            pl.BlockSpec((2, MB, U), lambda i: (0, off + i, 0)),
            pl.BlockSpec((3 * U, U), lambda i: (0, 0)),
            pl.BlockSpec((U,), lambda i: (0,)),
            pl.BlockSpec((U, U), lambda i: (0, 0)),
            pl.BlockSpec((U,), lambda i: (0,)),
        ],
        out_specs=pl.BlockSpec((MB, U), lambda i: (i, 0)),
        out_shape=jax.ShapeDtypeStruct((n, U), jnp.float32),
    )(h, parts, parts, w1, b1, w2, b2)


def _mlp_dn(h, parts, w1, b1, w2, b2, n):
    def body(h_ref, p_ref, w1_ref, b1_ref, w2_ref, b2_ref, o_ref):
        hv = h_ref[...]
        hd = p_ref[0] + p_ref[1]
        x = jnp.concatenate([hv, hd], axis=-1)
        x1 = (jnp.dot(x, w1_ref[...], preferred_element_type=jnp.float32)
              + b1_ref[...][None, :])
        t = jnp.tanh(x1)
        o_ref[...] = (jnp.dot(t, w2_ref[...], preferred_element_type=jnp.float32)
                      + b2_ref[...][None, :])
    return pl.pallas_call(
        body,
        grid=(pl.cdiv(n, MB),),
        in_specs=[
            pl.BlockSpec((MB, U), lambda i: (i, 0)),
            pl.BlockSpec((2, MB, U), lambda i: (0, i, 0)),
            pl.BlockSpec((2 * U, U), lambda i: (0, 0)),
            pl.BlockSpec((U,), lambda i: (0,)),
            pl.BlockSpec((U, U), lambda i: (0, 0)),
            pl.BlockSpec((U,), lambda i: (0,)),
        ],
        out_specs=pl.BlockSpec((MB, U), lambda i: (i, 0)),
        out_shape=jax.ShapeDtypeStruct((n, U), jnp.float32),
    )(h, parts, w1, b1, w2, b2)


def _pad_edges(ids, dump, epad):
    pad = epad - ids.shape[0]
    if pad == 0:
        return ids
    return jnp.concatenate([ids, jnp.full((pad,), dump, jnp.int32)])


def _epad(e):
    q = NW * CHUNK
    return ((e + q - 1) // q) * q


def kernel(h1, h2, h3, h4,
           e_up_2_0, e_up_2_1, e_up_3_0, e_up_3_1, e_up_4_0, e_up_4_1,
           e_down_4_0, e_down_4_1, e_down_3_0, e_down_3_1,
           e_down_2_0, e_down_2_1,
           up2_W1, up2_b1, up2_W2, up2_b2,
           up3_W1, up3_b1, up3_W2, up3_b2,
           up4_W1, up4_b1, up4_W2, up4_b2,
           dn1_W1, dn1_b1, dn1_W2, dn1_b2,
           dn2_W1, dn2_b1, dn2_W2, dn2_b2,
           dn3_W1, dn3_b1, dn3_W2, dn3_b2):
    hs = {1: h1, 2: h2, 3: h3, 4: h4}
    up_e = {2: (e_up_2_0, e_up_2_1), 3: (e_up_3_0, e_up_3_1),
            4: (e_up_4_0, e_up_4_1)}
    dn_e = {4: (e_down_4_0, e_down_4_1), 3: (e_down_3_0, e_down_3_1),
            2: (e_down_2_0, e_down_2_1)}
    up_w = {2: (up2_W1, up2_b1, up2_W2, up2_b2),
            3: (up3_W1, up3_b1, up3_W2, up3_b2),
            4: (up4_W1, up4_b1, up4_W2, up4_b2)}
    dn_w = {1: (dn1_W1, dn1_b1, dn1_W2, dn1_b2),
            2: (dn2_W1, dn2_b1, dn2_W2, dn2_b2),
            3: (dn3_W1, dn3_b1, dn3_W2, dn3_b2)}

    for g in (2, 3, 4):
        e0, e1 = up_e[g]
        ng = hs[g].shape[0]
        ngp = _pad_rows(ng)
        e = e0.shape[1]
        ep = _epad(e)
        src = jnp.concatenate([_pad_edges(e0[0], 0, ep),
                               _pad_edges(e1[0], 0, ep)])
        dst = jnp.concatenate([_pad_edges(e0[1], ng, ep),
                               _pad_edges(e1[1] + ngp, ngp + ng, ep)])
        parts = _make_agg(hs[g - 1].shape[0], 2 * ngp, 2 * ep)(
            hs[g - 1], src, dst)
        hs[g] = _mlp_up(hs[g], parts, *up_w[g], ng, ngp)

    for g in (4, 3, 2):
        e0, e1 = dn_e[g]
        nd = hs[g - 1].shape[0]
        ndp = _pad_rows(nd)
        e = e0.shape[1]
        ep = _epad(e)
        src = jnp.concatenate([_pad_edges(e0[0], 0, ep),
                               _pad_edges(e1[0], 0, ep)])
        dst = jnp.concatenate([_pad_edges(e0[1], nd, ep),
                               _pad_edges(e1[1], nd, ep)])
        parts = _make_agg(hs[g].shape[0], ndp, 2 * ep)(hs[g], src, dst)
        hs[g - 1] = _mlp_dn(hs[g - 1], parts, *dn_w[g - 1], nd)

    return hs[1], hs[2], hs[3], hs[4]
